# baseline (speedup 1.0000x reference)
import functools

import jax
import jax.numpy as jnp
import numpy as np
from jax import lax
from jax.experimental import pallas as pl
from jax.experimental.pallas import tpu as pltpu
from jax.experimental.pallas import tpu_sc as plsc

_N = 100000
_D = 128
_NC = 2
_NS = 16
_NW = _NC * _NS
_C = 128
_SB = 3
_NSUP = 8
_NFULL = _N // _C
_TAIL = _N - _NFULL * _C
_KMAX = 25
_NLONG = _NFULL - 24 * _NW

_PERM_CACHE: dict = {}


def _chunk_start(w: int):
    return _KMAX * w if w < _NLONG else 24 * w + _NLONG


def _perm_chunked():
    if "p" not in _PERM_CACHE:
        with jax.ensure_compile_time_eval():
            p = jax.random.permutation(jax.random.key(42), _N)
        p = np.asarray(p, dtype=np.int32)
        sup = np.zeros((_NW, _NSUP, _SB * _C), dtype=np.int32)
        last = np.zeros((_NW, _C), dtype=np.int32)
        for w in range(_NW):
            s = _chunk_start(w)
            sup[w] = p[_C * s: _C * (s + 24)].reshape(_NSUP, _SB * _C)
            if w < _NLONG:
                last[w] = p[_C * (s + 24): _C * (s + 25)]
        last[_NW - 1, :_TAIL] = p[_NFULL * _C:]
        _PERM_CACHE["p"] = (sup, last)
    return _PERM_CACHE["p"]


@functools.partial(
    pl.kernel,
    out_type=jax.ShapeDtypeStruct((_N, _D), jnp.float32),
    mesh=plsc.VectorSubcoreMesh(core_axis_name="c", subcore_axis_name="s"),
    scratch_types=[
        pltpu.VMEM((_NSUP, _SB * _C), jnp.int32),
        pltpu.VMEM((1, _C), jnp.int32),
        pltpu.VMEM((2, _SB * _C, _D), jnp.float32),
        pltpu.SemaphoreType.DMA,
        pltpu.SemaphoreType.DMA,
    ],
)
def _permute_rows(table_hbm, idxs_hbm, idxl_hbm, out_hbm,
                  idxs_v, idxl_v, bufs, gsem, ssem):
    wid = lax.axis_index("s") * _NC + lax.axis_index("c")
    start = jnp.where(wid < _NLONG, _KMAX * wid, 24 * wid + _NLONG)
    pltpu.sync_copy(idxs_hbm.at[wid], idxs_v)
    pltpu.sync_copy(idxl_hbm.at[pl.ds(wid, 1)], idxl_v)

    def gather_chunk(s, i):
        return pltpu.async_copy(
            table_hbm.at[idxs_v.at[s, pl.ds(i * _C, _C)]],
            bufs.at[s % 2, pl.ds(i * _C, _C)], gsem)

    def store_super(s):
        return pltpu.async_copy(
            bufs.at[s % 2],
            out_hbm.at[pl.ds((start + s * _SB) * _C, _SB * _C)], ssem)

    gathers = {0: [gather_chunk(0, i) for i in range(_SB)]}
    stores = {}
    for s in range(_NSUP):
        if s + 1 < _NSUP:
            if s - 1 in stores:
                stores.pop(s - 1).wait()
            gathers[s + 1] = [gather_chunk(s + 1, i) for i in range(_SB)]
        for g in gathers.pop(s):
            g.wait()
        stores[s] = store_super(s)
    for s in sorted(stores):
        stores.pop(s).wait()

    @pl.when(wid < _NLONG)
    def _():
        pltpu.async_copy(
            table_hbm.at[idxl_v.at[0]], bufs.at[0, pl.ds(0, _C)], gsem).wait()
        pltpu.async_copy(
            bufs.at[0, pl.ds(0, _C)],
            out_hbm.at[pl.ds((start + 24) * _C, _C)], ssem).wait()

    @pl.when(wid == _NW - 1)
    def _():
        pltpu.async_copy(
            table_hbm.at[idxl_v.at[0, pl.ds(0, _TAIL)]],
            bufs.at[0, pl.ds(0, _TAIL)], gsem).wait()
        pltpu.async_copy(
            bufs.at[0, pl.ds(0, _TAIL)],
            out_hbm.at[pl.ds(_NFULL * _C, _TAIL)], ssem).wait()


def kernel(data):
    x = data.reshape(_N, _D)
    sup, last = _perm_chunked()
    out = _permute_rows(x, jnp.asarray(sup), jnp.asarray(last))
    return out.reshape(_N * _D)


# speedup vs baseline: 1.0146x; 1.0146x over previous
import functools

import jax
import jax.numpy as jnp
import numpy as np
from jax import lax
from jax.experimental import pallas as pl
from jax.experimental.pallas import tpu as pltpu
from jax.experimental.pallas import tpu_sc as plsc

_N = 100000
_D = 128
_NC = 2
_NS = 16
_NW = _NC * _NS
_C = 128
_SB = 3
_NSUP = 8
_FULL = _SB * _C * _NSUP
_PL = 56
_PS = 48
_NLONG = 20

_PERM_CACHE: dict = {}


def _row_start(w: int) -> int:
    return _FULL * w + _PL * min(w, _NLONG) + _PS * max(0, w - _NLONG)


def _perm_chunked():
    if "p" not in _PERM_CACHE:
        with jax.ensure_compile_time_eval():
            p = jax.random.permutation(jax.random.key(42), _N)
        p = np.asarray(p, dtype=np.int32)
        sup = np.zeros((_NW, _NSUP, _SB * _C), dtype=np.int32)
        part = np.zeros((_NW, 64), dtype=np.int32)
        for w in range(_NW):
            r = _row_start(w)
            sup[w] = p[r: r + _FULL].reshape(_NSUP, _SB * _C)
            n = _PL if w < _NLONG else _PS
            part[w, :n] = p[r + _FULL: r + _FULL + n]
        _PERM_CACHE["p"] = (sup, part)
    return _PERM_CACHE["p"]


@functools.partial(
    pl.kernel,
    out_type=jax.ShapeDtypeStruct((_N, _D), jnp.float32),
    mesh=plsc.VectorSubcoreMesh(core_axis_name="c", subcore_axis_name="s"),
    scratch_types=[
        pltpu.VMEM((_NSUP, _SB * _C), jnp.int32),
        pltpu.VMEM((1, 64), jnp.int32),
        pltpu.VMEM((2, _SB * _C, _D), jnp.float32),
        pltpu.SemaphoreType.DMA,
        pltpu.SemaphoreType.DMA,
    ],
)
def _permute_rows(table_hbm, idxs_hbm, idxp_hbm, out_hbm,
                  idxs_v, idxp_v, bufs, gsem, ssem):
    wid = lax.axis_index("s") * _NC + lax.axis_index("c")
    base = 8 * jnp.where(wid < _NLONG,
                         wid * (_FULL + _PL) // 8,
                         (wid * (_FULL + _PS) + _NLONG * (_PL - _PS)) // 8)
    pltpu.sync_copy(idxs_hbm.at[wid], idxs_v)
    pltpu.sync_copy(idxp_hbm.at[pl.ds(wid, 1)], idxp_v)

    def gather_chunk(s, i):
        return pltpu.async_copy(
            table_hbm.at[idxs_v.at[s, pl.ds(i * _C, _C)]],
            bufs.at[s % 2, pl.ds(i * _C, _C)], gsem)

    def store_super(s):
        return pltpu.async_copy(
            bufs.at[s % 2],
            out_hbm.at[pl.ds(base + s * _SB * _C, _SB * _C)], ssem)

    gathers = {0: [gather_chunk(0, i) for i in range(_SB)]}
    stores = {}
    for s in range(_NSUP):
        if s + 1 < _NSUP:
            if s - 1 in stores:
                stores.pop(s - 1).wait()
            gathers[s + 1] = [gather_chunk(s + 1, i) for i in range(_SB)]
        for g in gathers.pop(s):
            g.wait()
        stores[s] = store_super(s)
    for s in sorted(stores):
        stores.pop(s).wait()

    def do_partial(n):
        pltpu.async_copy(
            table_hbm.at[idxp_v.at[0, pl.ds(0, n)]],
            bufs.at[0, pl.ds(0, n)], gsem).wait()
        pltpu.async_copy(
            bufs.at[0, pl.ds(0, n)],
            out_hbm.at[pl.ds(base + _FULL, n)], ssem).wait()

    @pl.when(wid < _NLONG)
    def _():
        do_partial(_PL)

    @pl.when(wid >= _NLONG)
    def _():
        do_partial(_PS)


def kernel(data):
    x = data.reshape(_N, _D)
    sup, part = _perm_chunked()
    out = _permute_rows(x, jnp.asarray(sup), jnp.asarray(part))
    return out.reshape(_N * _D)
